# baseline (speedup 1.0000x reference)
import functools

import jax
import jax.numpy as jnp
from jax import lax
from jax.experimental import pallas as pl
from jax.experimental.pallas import tpu as pltpu
from jax.experimental.pallas import tpu_sc as plsc

VOCAB = 1000
D = 1000
B = 1024
T = 200
NTOK = B * T
NW = 32
BPW = NTOK // NW
CHUNK = 40
NCHUNK = BPW // CHUNK


def _gather_body(table_hbm, idx_hbm, out_hbm, idx_v, rows_v, sem):
    c = lax.axis_index("c")
    s = lax.axis_index("s")
    wid = s * 2 + c
    base = wid * BPW
    pltpu.sync_copy(idx_hbm.at[pl.ds(base, BPW)], idx_v)

    def body(g, carry):
        off = g * CHUNK
        pltpu.async_copy(
            table_hbm.at[idx_v.at[pl.ds(off, CHUNK)]], rows_v, sem
        ).wait()
        pltpu.sync_copy(rows_v, out_hbm.at[pl.ds(base + off, CHUNK)])
        return carry

    lax.fori_loop(0, NCHUNK, body, 0)


@jax.jit
def kernel(idx, token_embedding):
    idx_flat = idx.reshape(-1)
    mesh = plsc.VectorSubcoreMesh(core_axis_name="c", subcore_axis_name="s")
    out = pl.kernel(
        _gather_body,
        out_type=jax.ShapeDtypeStruct((NTOK, D), jnp.float32),
        mesh=mesh,
        scratch_types=[
            pltpu.VMEM((BPW,), jnp.int32),
            pltpu.VMEM((CHUNK, D), jnp.float32),
            pltpu.SemaphoreType.DMA,
        ],
        compiler_params=pltpu.CompilerParams(use_tc_tiling_on_sc=False),
    )(token_embedding, idx_flat)
    return out.reshape(B, T, D)


# speedup vs baseline: 1.4942x; 1.4942x over previous
import functools

import jax
import jax.numpy as jnp
from jax import lax
from jax.experimental import pallas as pl
from jax.experimental.pallas import tpu as pltpu
from jax.experimental.pallas import tpu_sc as plsc

VOCAB = 1000
D = 1000
DPAD = 1024
B = 1024
T = 200
NTOK = B * T
NW = 32
BPW = NTOK // NW
CHUNK = 40
NCHUNK = BPW // CHUNK
MAIN = 896
TAIL = D - MAIN


def _fill_tail(rows_v, tail_v, r):
    for j in range(TAIL // 16):
        tail_v[r, pl.ds(16 * j, 16)] = rows_v[r, pl.ds(MAIN + 16 * j, 16)]
    if TAIL % 16:
        tail_v[r, pl.ds(TAIL - 16, 16)] = rows_v[r, pl.ds(MAIN + TAIL - 16, 16)]


def _gather_body(table_hbm, idx_hbm, out_hbm, idx_v, rows_v, tail_v, sem):
    c = lax.axis_index("c")
    s = lax.axis_index("s")
    wid = s * 2 + c
    base = wid * BPW
    pltpu.sync_copy(idx_hbm.at[pl.ds(base, BPW)], idx_v)

    def body(g, carry):
        off = g * CHUNK
        pltpu.async_copy(
            table_hbm.at[idx_v.at[pl.ds(off, CHUNK)]], rows_v, sem
        ).wait()

        def tail_row(r, carry2):
            _fill_tail(rows_v, tail_v, r)
            return carry2

        lax.fori_loop(0, CHUNK, tail_row, 0)
        pltpu.sync_copy(
            rows_v.at[:, pl.ds(0, MAIN)],
            out_hbm.at[pl.ds(base + off, CHUNK), pl.ds(0, MAIN)],
        )
        pltpu.sync_copy(
            tail_v,
            out_hbm.at[pl.ds(base + off, CHUNK), pl.ds(MAIN, TAIL)],
        )
        return carry

    lax.fori_loop(0, NCHUNK, body, 0)


@jax.jit
def kernel(idx, token_embedding):
    idx_flat = idx.reshape(-1)
    table_pad = jnp.pad(token_embedding, ((0, 0), (0, DPAD - D)))
    mesh = plsc.VectorSubcoreMesh(core_axis_name="c", subcore_axis_name="s")
    out = pl.kernel(
        _gather_body,
        out_type=jax.ShapeDtypeStruct((NTOK, D), jnp.float32),
        mesh=mesh,
        scratch_types=[
            pltpu.VMEM((BPW,), jnp.int32),
            pltpu.VMEM((CHUNK, DPAD), jnp.float32),
            pltpu.VMEM((CHUNK, TAIL), jnp.float32),
            pltpu.SemaphoreType.DMA,
        ],
    )(table_pad, idx_flat)
    return out.reshape(B, T, D)


# speedup vs baseline: 1.7113x; 1.1453x over previous
import functools

import jax
import jax.numpy as jnp
from jax import lax
from jax.experimental import pallas as pl
from jax.experimental.pallas import tpu as pltpu
from jax.experimental.pallas import tpu_sc as plsc

VOCAB = 1000
D = 1000
DPAD = 1024
B = 1024
T = 200
NTOK = B * T
NW = 32
BPW = NTOK // NW
CHUNK = 40
NCHUNK = BPW // CHUNK
MAIN = 896
TAIL = D - MAIN


def _fill_tail(rows_v, tail_v, b, r):
    for j in range(TAIL // 16):
        tail_v[b, r, pl.ds(16 * j, 16)] = rows_v[b, r, pl.ds(MAIN + 16 * j, 16)]
    if TAIL % 16:
        tail_v[b, r, pl.ds(TAIL - 16, 16)] = rows_v[
            b, r, pl.ds(MAIN + TAIL - 16, 16)
        ]


def _gather_body(table_hbm, idx_hbm, out_hbm, idx_v, rows_v, tail_v,
                 sem_g, sem_o):
    c_ax = lax.axis_index("c")
    s_ax = lax.axis_index("s")
    wid = s_ax * 2 + c_ax
    base = wid * BPW
    pltpu.sync_copy(idx_hbm.at[pl.ds(base, BPW)], idx_v)

    def gather(c, b):
        return pltpu.make_async_copy(
            table_hbm.at[idx_v.at[pl.ds(c * CHUNK, CHUNK)]],
            rows_v.at[b],
            sem_g.at[b],
        )

    def main_copy(c, b):
        return pltpu.make_async_copy(
            rows_v.at[b, :, pl.ds(0, MAIN)],
            out_hbm.at[pl.ds(base + c * CHUNK, CHUNK), pl.ds(0, MAIN)],
            sem_o.at[b],
        )

    def tail_copy(c, b):
        return pltpu.make_async_copy(
            tail_v.at[b],
            out_hbm.at[pl.ds(base + c * CHUNK, CHUNK), pl.ds(MAIN, TAIL)],
            sem_o.at[b],
        )

    gather(0, 0).start()

    def body(c, carry):
        b = lax.rem(c, 2)
        nb = 1 - b
        gather(c, b).wait()
        main_copy(c, b).start()

        def tail_row(r, carry2):
            _fill_tail(rows_v, tail_v, b, r)
            return carry2

        lax.fori_loop(0, CHUNK, tail_row, 0)
        tail_copy(c, b).start()

        @pl.when(c >= 1)
        def _():
            main_copy(c - 1, nb).wait()
            tail_copy(c - 1, nb).wait()

        @pl.when(c + 1 < NCHUNK)
        def _():
            gather(c + 1, nb).start()

        return carry

    lax.fori_loop(0, NCHUNK, body, 0)
    last = NCHUNK - 1
    main_copy(last, last % 2).wait()
    tail_copy(last, last % 2).wait()


@jax.jit
def kernel(idx, token_embedding):
    idx_flat = idx.reshape(-1)
    table_pad = jnp.pad(token_embedding, ((0, 0), (0, DPAD - D)))
    mesh = plsc.VectorSubcoreMesh(core_axis_name="c", subcore_axis_name="s")
    out = pl.kernel(
        _gather_body,
        out_type=jax.ShapeDtypeStruct((NTOK, D), jnp.float32),
        mesh=mesh,
        scratch_types=[
            pltpu.VMEM((BPW,), jnp.int32),
            pltpu.VMEM((2, CHUNK, DPAD), jnp.float32),
            pltpu.VMEM((2, CHUNK, TAIL), jnp.float32),
            pltpu.SemaphoreType.DMA((2,)),
            pltpu.SemaphoreType.DMA((2,)),
        ],
    )(table_pad, idx_flat)
    return out.reshape(B, T, D)


# speedup vs baseline: 3.0605x; 1.7884x over previous
import functools

import jax
import jax.numpy as jnp
from jax import lax
from jax.experimental import pallas as pl
from jax.experimental.pallas import tpu as pltpu
from jax.experimental.pallas import tpu_sc as plsc

VOCAB = 1000
D = 1000
B = 1024
T = 200
NW = 32
BT = B // 128
NFRAG = T * BT
FPW = NFRAG // NW
DSLAB = 40
NSLAB = D // DSLAB
SLABW = DSLAB * VOCAB
NITER = NSLAB * FPW


def _body(tableT_hbm, idxT_hbm, out_hbm, idx_v, slab_v, frag_v, sem_o):
    c_ax = lax.axis_index("c")
    s_ax = lax.axis_index("s")
    wid = s_ax * 2 + c_ax

    def stage_idx(k, carry):
        f = wid + NW * k
        pltpu.sync_copy(
            idxT_hbm.at[pl.ds(f * 128, 128)],
            idx_v.at[pl.ds(k * 128, 128)],
        )
        return carry

    lax.fori_loop(0, FPW, stage_idx, 0)

    def frag_dma(k, s, p):
        f = wid + NW * k
        t = f // BT
        bt = lax.rem(f, BT)
        return pltpu.make_async_copy(
            frag_v.at[p],
            out_hbm.at[t, pl.ds(s * DSLAB, DSLAB), pl.ds(bt * 128, 128)],
            sem_o.at[p],
        )

    def body(i, carry):
        s = i // FPW
        k = lax.rem(i, FPW)
        p = lax.rem(i, 2)

        @pl.when(k == 0)
        def _():
            pltpu.sync_copy(
                tableT_hbm.at[pl.ds(s * SLABW, SLABW)], slab_v
            )

        @pl.when(i >= 2)
        def _():
            frag_dma(k, s, p).wait()

        tok = [
            idx_v[pl.ds(k * 128 + 16 * j, 16)] for j in range(8)
        ]

        def drow(d, iv):
            got = [plsc.load_gather(slab_v, [iv[j]]) for j in range(8)]
            for j in range(8):
                frag_v[p, d, pl.ds(16 * j, 16)] = got[j]
            return tuple(v + 1000 for v in iv)

        lax.fori_loop(0, DSLAB, drow, tuple(tok))
        frag_dma(k, s, p).start()
        return carry

    lax.fori_loop(0, NITER, body, 0)
    frag_dma(FPW - 1, NSLAB - 1, 0).wait()
    frag_dma(FPW - 1, NSLAB - 1, 1).wait()


@jax.jit
def kernel(idx, token_embedding):
    idxT_flat = idx.T.reshape(-1)
    tableT_flat = token_embedding.T.reshape(-1)
    mesh = plsc.VectorSubcoreMesh(core_axis_name="c", subcore_axis_name="s")
    out = pl.kernel(
        _body,
        out_type=jax.ShapeDtypeStruct((T, D, B), jnp.float32),
        mesh=mesh,
        scratch_types=[
            pltpu.VMEM((FPW * 128,), jnp.int32),
            pltpu.VMEM((SLABW,), jnp.float32),
            pltpu.VMEM((2, DSLAB, 128), jnp.float32),
            pltpu.SemaphoreType.DMA((2,)),
        ],
        compiler_params=pltpu.CompilerParams(needs_layout_passes=False),
    )(tableT_flat, idxT_flat)
    return out.transpose(2, 0, 1)


# speedup vs baseline: 3.3454x; 1.0931x over previous
import functools

import jax
import jax.numpy as jnp
from jax import lax
from jax.experimental import pallas as pl
from jax.experimental.pallas import tpu as pltpu
from jax.experimental.pallas import tpu_sc as plsc

VOCAB = 1000
D = 1000
B = 1024
T = 200
NW = 32
BT = B // 128
NFRAG = T * BT
FPW = NFRAG // NW
DSLAB = 40
NSLAB = D // DSLAB
SLABW = DSLAB * VOCAB
NITER = NSLAB * FPW


def _body(tableT_hbm, idxT_hbm, out_hbm, idx_v, slab_v, frag_v, sem_o,
          sem_s):
    c_ax = lax.axis_index("c")
    s_ax = lax.axis_index("s")
    wid = s_ax * 2 + c_ax

    def stage_idx(k, carry):
        f = wid + NW * k
        pltpu.sync_copy(
            idxT_hbm.at[pl.ds(f * 128, 128)],
            idx_v.at[pl.ds(k * 128, 128)],
        )
        return carry

    lax.fori_loop(0, FPW, stage_idx, 0)

    def frag_dma(k, s, p):
        f = wid + NW * k
        t = f // BT
        bt = lax.rem(f, BT)
        return pltpu.make_async_copy(
            frag_v.at[p],
            out_hbm.at[t, pl.ds(s * DSLAB, DSLAB), pl.ds(bt * 128, 128)],
            sem_o.at[p],
        )

    def stage_dma(s, sb):
        return pltpu.make_async_copy(
            tableT_hbm.at[pl.ds(s * SLABW, SLABW)],
            slab_v.at[pl.ds(sb * SLABW, SLABW)],
            sem_s.at[sb],
        )

    stage_dma(0, 0).start()

    def body(i, carry):
        s = i // FPW
        k = lax.rem(i, FPW)
        p = lax.rem(i, 2)
        sb = lax.rem(s, 2)

        @pl.when(k == 0)
        def _():
            stage_dma(s, sb).wait()

            @pl.when(s + 1 < NSLAB)
            def _():
                stage_dma(s + 1, 1 - sb).start()

        @pl.when(i >= 2)
        def _():
            frag_dma(k, s, p).wait()

        off = sb * SLABW
        tok = [
            idx_v[pl.ds(k * 128 + 16 * j, 16)] + off for j in range(8)
        ]

        def drow(d, iv):
            got = [plsc.load_gather(slab_v, [iv[j]]) for j in range(8)]
            for j in range(8):
                frag_v[p, d, pl.ds(16 * j, 16)] = got[j]
            return tuple(v + 1000 for v in iv)

        lax.fori_loop(0, DSLAB, drow, tuple(tok))
        frag_dma(k, s, p).start()
        return carry

    lax.fori_loop(0, NITER, body, 0)
    frag_dma(FPW - 1, NSLAB - 1, 0).wait()
    frag_dma(FPW - 1, NSLAB - 1, 1).wait()


@jax.jit
def kernel(idx, token_embedding):
    idxT_flat = idx.T.reshape(-1)
    tableT_flat = token_embedding.T.reshape(-1)
    mesh = plsc.VectorSubcoreMesh(core_axis_name="c", subcore_axis_name="s")
    out = pl.kernel(
        _body,
        out_type=jax.ShapeDtypeStruct((T, D, B), jnp.float32),
        mesh=mesh,
        scratch_types=[
            pltpu.VMEM((FPW * 128,), jnp.int32),
            pltpu.VMEM((2 * SLABW,), jnp.float32),
            pltpu.VMEM((2, DSLAB, 128), jnp.float32),
            pltpu.SemaphoreType.DMA((2,)),
            pltpu.SemaphoreType.DMA((2,)),
        ],
        compiler_params=pltpu.CompilerParams(needs_layout_passes=False),
    )(tableT_flat, idxT_flat)
    return out.transpose(2, 0, 1)


# speedup vs baseline: 3.5252x; 1.0537x over previous
import functools

import jax
import jax.numpy as jnp
from jax import lax
from jax.experimental import pallas as pl
from jax.experimental.pallas import tpu as pltpu
from jax.experimental.pallas import tpu_sc as plsc

VOCAB = 1000
D = 1000
B = 1024
T = 200
NW = 32
BT = B // 128
NFRAG = T * BT
FPW = NFRAG // NW
DSLAB = 40
NSLAB = D // DSLAB
SLABW = DSLAB * VOCAB
NITER = NSLAB * FPW


def _body(tableT_hbm, idxT_hbm, out_hbm, idx_v, slab_v, frag_v, sem_o,
          sem_s):
    c_ax = lax.axis_index("c")
    s_ax = lax.axis_index("s")
    wid = s_ax * 2 + c_ax

    def stage_idx(k, carry):
        f = wid + NW * k
        pltpu.sync_copy(
            idxT_hbm.at[pl.ds(f * 128, 128)],
            idx_v.at[pl.ds(k * 128, 128)],
        )
        return carry

    lax.fori_loop(0, FPW, stage_idx, 0)

    def frag_dma(k, s, p):
        f = wid + NW * k
        t = f // BT
        bt = lax.rem(f, BT)
        return pltpu.make_async_copy(
            frag_v.at[p],
            out_hbm.at[t, pl.ds(s * DSLAB, DSLAB), pl.ds(bt * 128, 128)],
            sem_o.at[p],
        )

    def stage_dma(s, sb):
        return pltpu.make_async_copy(
            tableT_hbm.at[pl.ds(s * SLABW, SLABW)],
            slab_v.at[pl.ds(sb * SLABW, SLABW)],
            sem_s.at[sb],
        )

    stage_dma(0, 0).start()

    def body(i, carry):
        s = i // FPW
        k = lax.rem(i, FPW)
        p = lax.rem(i, 2)
        sb = lax.rem(s, 2)

        @pl.when(k == 0)
        def _():
            stage_dma(s, sb).wait()

            @pl.when(s + 1 < NSLAB)
            def _():
                stage_dma(s + 1, 1 - sb).start()

        @pl.when(i >= 2)
        def _():
            frag_dma(k, s, p).wait()

        off = sb * SLABW
        tok = [
            idx_v[pl.ds(k * 128 + 16 * j, 16)] + off for j in range(8)
        ]

        def drow(h, iv):
            d = 2 * h
            got = [plsc.load_gather(slab_v, [iv[j]]) for j in range(8)]
            iv2 = tuple(v + VOCAB for v in iv)
            got2 = [plsc.load_gather(slab_v, [iv2[j]]) for j in range(8)]
            for j in range(8):
                frag_v[p, d, pl.ds(16 * j, 16)] = got[j]
            for j in range(8):
                frag_v[p, d + 1, pl.ds(16 * j, 16)] = got2[j]
            return tuple(v + VOCAB for v in iv2)

        lax.fori_loop(0, DSLAB // 2, drow, tuple(tok))
        frag_dma(k, s, p).start()
        return carry

    lax.fori_loop(0, NITER, body, 0)
    frag_dma(FPW - 1, NSLAB - 1, 0).wait()
    frag_dma(FPW - 1, NSLAB - 1, 1).wait()


@jax.jit
def kernel(idx, token_embedding):
    idxT_flat = idx.T.reshape(-1)
    tableT_flat = token_embedding.T.reshape(-1)
    mesh = plsc.VectorSubcoreMesh(core_axis_name="c", subcore_axis_name="s")
    out = pl.kernel(
        _body,
        out_type=jax.ShapeDtypeStruct((T, D, B), jnp.float32),
        mesh=mesh,
        scratch_types=[
            pltpu.VMEM((FPW * 128,), jnp.int32),
            pltpu.VMEM((2 * SLABW,), jnp.float32),
            pltpu.VMEM((2, DSLAB, 128), jnp.float32),
            pltpu.SemaphoreType.DMA((2,)),
            pltpu.SemaphoreType.DMA((2,)),
        ],
        compiler_params=pltpu.CompilerParams(needs_layout_passes=False),
    )(tableT_flat, idxT_flat)
    return out.transpose(2, 0, 1)
